# baseline (speedup 1.0000x reference)
import functools

import jax
import jax.numpy as jnp
from jax import lax
from jax.experimental import pallas as pl
from jax.experimental.pallas import tpu as pltpu
from jax.experimental.pallas import tpu_sc as plsc

BATCH = 4096
EMBED_DIM = 64
TOPIC_SIZE = 128
TABLE_ROWS = 100000
PAIR_ROWS = TABLE_ROWS // 2
PAIR_DIM = 2 * EMBED_DIM

_info = plsc.get_sparse_core_info()
_NC, _NS = _info.num_cores, _info.num_subcores
_NW = _NC * _NS
_BPW = BATCH // _NW


def _sc_body(users_hbm, items_hbm, ut2_hbm, it2_hbm, eu2_hbm, ei2_hbm,
             uid_v, iid_v, utix_v, itix_v, urows_v, irows_v, sem):
    wid = lax.axis_index("s") * _NC + lax.axis_index("c")
    base = wid * _BPW
    pltpu.sync_copy(users_hbm.at[pl.ds(base, _BPW)], uid_v)
    pltpu.sync_copy(items_hbm.at[pl.ds(base, _BPW)], iid_v)
    for j in range(_BPW // 16):
        utix_v[pl.ds(j * 16, 16)] = uid_v[pl.ds(j * 16, 16)] >> 1
        itix_v[pl.ds(j * 16, 16)] = iid_v[pl.ds(j * 16, 16)] >> 1
    cu = pltpu.async_copy(ut2_hbm.at[utix_v], urows_v, sem)
    ci = pltpu.async_copy(it2_hbm.at[itix_v], irows_v, sem)
    cu.wait()
    ci.wait()
    pltpu.sync_copy(urows_v, eu2_hbm.at[pl.ds(base, _BPW)])
    pltpu.sync_copy(irows_v, ei2_hbm.at[pl.ds(base, _BPW)])


_sc_gather = functools.partial(
    pl.kernel,
    mesh=plsc.VectorSubcoreMesh(core_axis_name="c", subcore_axis_name="s"),
    out_type=[
        jax.ShapeDtypeStruct((BATCH, PAIR_DIM), jnp.float32),
        jax.ShapeDtypeStruct((BATCH, PAIR_DIM), jnp.float32),
    ],
    scratch_types=[
        pltpu.VMEM((_BPW,), jnp.int32),
        pltpu.VMEM((_BPW,), jnp.int32),
        pltpu.VMEM((_BPW,), jnp.int32),
        pltpu.VMEM((_BPW,), jnp.int32),
        pltpu.VMEM((_BPW, PAIR_DIM), jnp.float32),
        pltpu.VMEM((_BPW, PAIR_DIM), jnp.float32),
        pltpu.SemaphoreType.DMA,
    ],
)(_sc_body)


def _tc_combine_body(users_ref, items_ref, ctx_ref, topic_ref,
                     eu2_ref, ei2_ref, out_ref):
    pu = (users_ref[...] & 1)[:, None] == 1
    pi = (items_ref[...] & 1)[:, None] == 1
    eu = jnp.where(pu, eu2_ref[:, EMBED_DIM:], eu2_ref[:, :EMBED_DIM])
    ei = jnp.where(pi, ei2_ref[:, EMBED_DIM:], ei2_ref[:, :EMBED_DIM])
    ctx = jnp.dot(ctx_ref[...], topic_ref[...],
                  preferred_element_type=jnp.float32)
    out_ref[...] = jnp.sum(eu * (ctx + ei), axis=1)


def _tc_combine(users, items, item_ctx, topic_table, eu2, ei2):
    nblk = 8
    bs = BATCH // nblk
    return pl.pallas_call(
        _tc_combine_body,
        grid=(nblk,),
        in_specs=[
            pl.BlockSpec((bs,), lambda i: (i,)),
            pl.BlockSpec((bs,), lambda i: (i,)),
            pl.BlockSpec((bs, TOPIC_SIZE), lambda i: (i, 0)),
            pl.BlockSpec((TOPIC_SIZE, EMBED_DIM), lambda i: (0, 0)),
            pl.BlockSpec((bs, PAIR_DIM), lambda i: (i, 0)),
            pl.BlockSpec((bs, PAIR_DIM), lambda i: (i, 0)),
        ],
        out_specs=pl.BlockSpec((bs,), lambda i: (i,)),
        out_shape=jax.ShapeDtypeStruct((BATCH,), jnp.float32),
    )(users, items, item_ctx, topic_table, eu2, ei2)


@jax.jit
def kernel(users, items, item_ctx, user_table, item_table, topic_table):
    ut2 = user_table.reshape(PAIR_ROWS, PAIR_DIM)
    it2 = item_table.reshape(PAIR_ROWS, PAIR_DIM)
    eu2, ei2 = _sc_gather(users, items, ut2, it2)
    return _tc_combine(users, items, item_ctx, topic_table, eu2, ei2)


# speedup vs baseline: 1.6116x; 1.6116x over previous
import functools

import jax
import jax.numpy as jnp
from jax import lax
from jax.experimental import pallas as pl
from jax.experimental.pallas import tpu as pltpu
from jax.experimental.pallas import tpu_sc as plsc

BATCH = 4096
EMBED_DIM = 64
TOPIC_SIZE = 128
TABLE_ROWS = 100000
PAIR_DIM = 2 * EMBED_DIM
SUP = 4096
HALF = SUP // 2
NSUP = -(-TABLE_ROWS // SUP)
PAIR_ROWS = NSUP * HALF

_info = plsc.get_sparse_core_info()
_NC, _NS = _info.num_cores, _info.num_subcores
_NW = _NC * _NS
_BPW = BATCH // _NW


def _tc_repack_body(t1_ref, t2_ref, out_ref):
    out_ref[...] = jnp.concatenate([t1_ref[...], t2_ref[...]], axis=0).T


def _tc_repack(table_t):
    return pl.pallas_call(
        _tc_repack_body,
        grid=(NSUP,),
        in_specs=[
            pl.BlockSpec((EMBED_DIM, HALF), lambda i: (0, 2 * i)),
            pl.BlockSpec((EMBED_DIM, HALF),
                         lambda i: (0, jnp.minimum(2 * i + 1,
                                                   TABLE_ROWS // HALF))),
        ],
        out_specs=pl.BlockSpec((HALF, PAIR_DIM), lambda i: (i, 0)),
        out_shape=jax.ShapeDtypeStruct((PAIR_ROWS, PAIR_DIM), jnp.float32),
    )(table_t, table_t)


def _sc_body(users_hbm, items_hbm, ut2_hbm, it2_hbm, eu2_hbm, ei2_hbm,
             uid_v, iid_v, utix_v, itix_v, urows_v, irows_v, sem):
    wid = lax.axis_index("s") * _NC + lax.axis_index("c")
    base = wid * _BPW
    pltpu.sync_copy(users_hbm.at[pl.ds(base, _BPW)], uid_v)
    pltpu.sync_copy(items_hbm.at[pl.ds(base, _BPW)], iid_v)
    for j in range(_BPW // 16):
        u = uid_v[pl.ds(j * 16, 16)]
        i = iid_v[pl.ds(j * 16, 16)]
        utix_v[pl.ds(j * 16, 16)] = ((u >> 12) << 11) | (u & (HALF - 1))
        itix_v[pl.ds(j * 16, 16)] = ((i >> 12) << 11) | (i & (HALF - 1))
    cu = pltpu.async_copy(ut2_hbm.at[utix_v], urows_v, sem)
    ci = pltpu.async_copy(it2_hbm.at[itix_v], irows_v, sem)
    cu.wait()
    ci.wait()
    pltpu.sync_copy(urows_v, eu2_hbm.at[pl.ds(base, _BPW)])
    pltpu.sync_copy(irows_v, ei2_hbm.at[pl.ds(base, _BPW)])


_sc_gather = functools.partial(
    pl.kernel,
    mesh=plsc.VectorSubcoreMesh(core_axis_name="c", subcore_axis_name="s"),
    out_type=[
        jax.ShapeDtypeStruct((BATCH, PAIR_DIM), jnp.float32),
        jax.ShapeDtypeStruct((BATCH, PAIR_DIM), jnp.float32),
    ],
    scratch_types=[
        pltpu.VMEM((_BPW,), jnp.int32),
        pltpu.VMEM((_BPW,), jnp.int32),
        pltpu.VMEM((_BPW,), jnp.int32),
        pltpu.VMEM((_BPW,), jnp.int32),
        pltpu.VMEM((_BPW, PAIR_DIM), jnp.float32),
        pltpu.VMEM((_BPW, PAIR_DIM), jnp.float32),
        pltpu.SemaphoreType.DMA,
    ],
)(_sc_body)


def _tc_combine_body(users_ref, items_ref, ctx_ref, topic_ref,
                     eu2_ref, ei2_ref, out_ref):
    pu = ((users_ref[...] >> 11) & 1)[:, None] == 1
    pi = ((items_ref[...] >> 11) & 1)[:, None] == 1
    eu = jnp.where(pu, eu2_ref[:, EMBED_DIM:], eu2_ref[:, :EMBED_DIM])
    ei = jnp.where(pi, ei2_ref[:, EMBED_DIM:], ei2_ref[:, :EMBED_DIM])
    ctx = jnp.dot(ctx_ref[...], topic_ref[...],
                  preferred_element_type=jnp.float32)
    out_ref[...] = jnp.sum(eu * (ctx + ei), axis=1)


def _tc_combine(users, items, item_ctx, topic_table, eu2, ei2):
    nblk = 8
    bs = BATCH // nblk
    return pl.pallas_call(
        _tc_combine_body,
        grid=(nblk,),
        in_specs=[
            pl.BlockSpec((bs,), lambda i: (i,)),
            pl.BlockSpec((bs,), lambda i: (i,)),
            pl.BlockSpec((bs, TOPIC_SIZE), lambda i: (i, 0)),
            pl.BlockSpec((TOPIC_SIZE, EMBED_DIM), lambda i: (0, 0)),
            pl.BlockSpec((bs, PAIR_DIM), lambda i: (i, 0)),
            pl.BlockSpec((bs, PAIR_DIM), lambda i: (i, 0)),
        ],
        out_specs=pl.BlockSpec((bs,), lambda i: (i,)),
        out_shape=jax.ShapeDtypeStruct((BATCH,), jnp.float32),
    )(users, items, item_ctx, topic_table, eu2, ei2)


@jax.jit
def kernel(users, items, item_ctx, user_table, item_table, topic_table):
    ut2 = _tc_repack(user_table.T)
    it2 = _tc_repack(item_table.T)
    eu2, ei2 = _sc_gather(users, items, ut2, it2)
    return _tc_combine(users, items, item_ctx, topic_table, eu2, ei2)


# speedup vs baseline: 1.6715x; 1.0372x over previous
import functools

import jax
import jax.numpy as jnp
from jax import lax
from jax.experimental import pallas as pl
from jax.experimental.pallas import tpu as pltpu
from jax.experimental.pallas import tpu_sc as plsc

BATCH = 4096
EMBED_DIM = 64
TOPIC_SIZE = 128
TABLE_ROWS = 100000
PAIR_DIM = 2 * EMBED_DIM
SUP = 4096
HALF = SUP // 2
NSUP = -(-TABLE_ROWS // SUP)
PAIR_ROWS = NSUP * HALF

_info = plsc.get_sparse_core_info()
_NC, _NS = _info.num_cores, _info.num_subcores
_NW = _NC * _NS
_BPW = BATCH // _NW


def _tc_repack_body(t1_ref, t2_ref, out_ref):
    out_ref[...] = jnp.concatenate([t1_ref[...], t2_ref[...]], axis=0).T


def _tc_repack(table_t):
    return pl.pallas_call(
        _tc_repack_body,
        grid=(NSUP,),
        in_specs=[
            pl.BlockSpec((EMBED_DIM, HALF), lambda i: (0, 2 * i)),
            pl.BlockSpec((EMBED_DIM, HALF),
                         lambda i: (0, jnp.minimum(2 * i + 1,
                                                   TABLE_ROWS // HALF))),
        ],
        out_specs=pl.BlockSpec((HALF, PAIR_DIM), lambda i: (i, 0)),
        out_shape=jax.ShapeDtypeStruct((PAIR_ROWS, PAIR_DIM), jnp.float32),
    )(table_t, table_t)


def _sc_body(idx_hbm, tab2_hbm, out2_hbm, idx_v, pix_v, rows_v, sem):
    wid = lax.axis_index("s") * _NC + lax.axis_index("c")
    base = wid * _BPW
    pltpu.sync_copy(idx_hbm.at[pl.ds(base, _BPW)], idx_v)
    for j in range(_BPW // 16):
        r = idx_v[pl.ds(j * 16, 16)]
        pix_v[pl.ds(j * 16, 16)] = ((r >> 12) << 11) | (r & (HALF - 1))
    pltpu.async_copy(tab2_hbm.at[pix_v], rows_v, sem).wait()
    pltpu.sync_copy(rows_v, out2_hbm.at[pl.ds(base, _BPW)])


_sc_gather = functools.partial(
    pl.kernel,
    mesh=plsc.VectorSubcoreMesh(core_axis_name="c", subcore_axis_name="s"),
    out_type=jax.ShapeDtypeStruct((BATCH, PAIR_DIM), jnp.float32),
    scratch_types=[
        pltpu.VMEM((_BPW,), jnp.int32),
        pltpu.VMEM((_BPW,), jnp.int32),
        pltpu.VMEM((_BPW, PAIR_DIM), jnp.float32),
        pltpu.SemaphoreType.DMA,
    ],
)(_sc_body)


def _tc_combine_body(users_ref, items_ref, ctx_ref, topic_ref,
                     eu2_ref, ei2_ref, out_ref):
    pu = ((users_ref[...] >> 11) & 1)[:, None] == 1
    pi = ((items_ref[...] >> 11) & 1)[:, None] == 1
    eu = jnp.where(pu, eu2_ref[:, EMBED_DIM:], eu2_ref[:, :EMBED_DIM])
    ei = jnp.where(pi, ei2_ref[:, EMBED_DIM:], ei2_ref[:, :EMBED_DIM])
    ctx = jnp.dot(ctx_ref[...], topic_ref[...],
                  preferred_element_type=jnp.float32)
    out_ref[...] = jnp.sum(eu * (ctx + ei), axis=1)


def _tc_combine(users, items, item_ctx, topic_table, eu2, ei2):
    nblk = 4
    bs = BATCH // nblk
    return pl.pallas_call(
        _tc_combine_body,
        grid=(nblk,),
        in_specs=[
            pl.BlockSpec((bs,), lambda i: (i,)),
            pl.BlockSpec((bs,), lambda i: (i,)),
            pl.BlockSpec((bs, TOPIC_SIZE), lambda i: (i, 0)),
            pl.BlockSpec((TOPIC_SIZE, EMBED_DIM), lambda i: (0, 0)),
            pl.BlockSpec((bs, PAIR_DIM), lambda i: (i, 0)),
            pl.BlockSpec((bs, PAIR_DIM), lambda i: (i, 0)),
        ],
        out_specs=pl.BlockSpec((bs,), lambda i: (i,)),
        out_shape=jax.ShapeDtypeStruct((BATCH,), jnp.float32),
    )(users, items, item_ctx, topic_table, eu2, ei2)


@jax.jit
def kernel(users, items, item_ctx, user_table, item_table, topic_table):
    ut2 = _tc_repack(user_table.T)
    eu2 = _sc_gather(users, ut2)
    it2 = _tc_repack(item_table.T)
    ei2 = _sc_gather(items, it2)
    return _tc_combine(users, items, item_ctx, topic_table, eu2, ei2)


# speedup vs baseline: 1.9989x; 1.1959x over previous
import functools

import jax
import jax.numpy as jnp
from jax import lax
from jax.experimental import pallas as pl
from jax.experimental.pallas import tpu as pltpu
from jax.experimental.pallas import tpu_sc as plsc

BATCH = 4096
EMBED_DIM = 64
TOPIC_SIZE = 128
TABLE_ROWS = 100000
PAIR_DIM = 2 * EMBED_DIM
SUP = 8192
HALF = SUP // 2
HBITS = 12
NSUP = -(-TABLE_ROWS // SUP)
PAIR_ROWS = NSUP * HALF

_info = plsc.get_sparse_core_info()
_NC, _NS = _info.num_cores, _info.num_subcores
_NW = _NC * _NS
_BPW = BATCH // _NW


def _tc_repack_body(t1_ref, t2_ref, out_ref):
    out_ref[...] = jnp.concatenate([t1_ref[...], t2_ref[...]], axis=0).T


def _tc_repack(table_t):
    return pl.pallas_call(
        _tc_repack_body,
        grid=(NSUP,),
        in_specs=[
            pl.BlockSpec((EMBED_DIM, HALF), lambda i: (0, 2 * i)),
            pl.BlockSpec((EMBED_DIM, HALF),
                         lambda i: (0, jnp.minimum(2 * i + 1,
                                                   TABLE_ROWS // HALF))),
        ],
        out_specs=pl.BlockSpec((HALF, PAIR_DIM), lambda i: (i, 0)),
        out_shape=jax.ShapeDtypeStruct((PAIR_ROWS, PAIR_DIM), jnp.float32),
    )(table_t, table_t)


def _sc_body(idx_hbm, tab2_hbm, out2_hbm, idx_v, pix_v, rows_v, sem):
    wid = lax.axis_index("s") * _NC + lax.axis_index("c")
    base = wid * _BPW
    pltpu.sync_copy(idx_hbm.at[pl.ds(base, _BPW)], idx_v)
    for j in range(_BPW // 16):
        r = idx_v[pl.ds(j * 16, 16)]
        pix_v[pl.ds(j * 16, 16)] = (
            ((r >> (HBITS + 1)) << HBITS) | (r & (HALF - 1)))
    pltpu.async_copy(tab2_hbm.at[pix_v], rows_v, sem).wait()
    pltpu.sync_copy(rows_v, out2_hbm.at[pl.ds(base, _BPW)])


_sc_gather = functools.partial(
    pl.kernel,
    mesh=plsc.VectorSubcoreMesh(core_axis_name="c", subcore_axis_name="s"),
    out_type=jax.ShapeDtypeStruct((BATCH, PAIR_DIM), jnp.float32),
    scratch_types=[
        pltpu.VMEM((_BPW,), jnp.int32),
        pltpu.VMEM((_BPW,), jnp.int32),
        pltpu.VMEM((_BPW, PAIR_DIM), jnp.float32),
        pltpu.SemaphoreType.DMA,
    ],
)(_sc_body)


def _tc_combine_body(users_ref, items_ref, ctx_ref, topic_ref,
                     eu2_ref, ei2_ref, out_ref):
    pu = ((users_ref[...] >> HBITS) & 1)[:, None] == 1
    pi = ((items_ref[...] >> HBITS) & 1)[:, None] == 1
    eu = jnp.where(pu, eu2_ref[:, EMBED_DIM:], eu2_ref[:, :EMBED_DIM])
    ei = jnp.where(pi, ei2_ref[:, EMBED_DIM:], ei2_ref[:, :EMBED_DIM])
    ctx = jnp.dot(ctx_ref[...], topic_ref[...],
                  preferred_element_type=jnp.float32)
    out_ref[...] = jnp.sum(eu * (ctx + ei), axis=1)


def _tc_combine(users, items, item_ctx, topic_table, eu2, ei2):
    nblk = 4
    bs = BATCH // nblk
    return pl.pallas_call(
        _tc_combine_body,
        grid=(nblk,),
        in_specs=[
            pl.BlockSpec((bs,), lambda i: (i,)),
            pl.BlockSpec((bs,), lambda i: (i,)),
            pl.BlockSpec((bs, TOPIC_SIZE), lambda i: (i, 0)),
            pl.BlockSpec((TOPIC_SIZE, EMBED_DIM), lambda i: (0, 0)),
            pl.BlockSpec((bs, PAIR_DIM), lambda i: (i, 0)),
            pl.BlockSpec((bs, PAIR_DIM), lambda i: (i, 0)),
        ],
        out_specs=pl.BlockSpec((bs,), lambda i: (i,)),
        out_shape=jax.ShapeDtypeStruct((BATCH,), jnp.float32),
    )(users, items, item_ctx, topic_table, eu2, ei2)


@jax.jit
def kernel(users, items, item_ctx, user_table, item_table, topic_table):
    ut2 = _tc_repack(user_table.T)
    eu2 = _sc_gather(users, ut2)
    it2 = _tc_repack(item_table.T)
    ei2 = _sc_gather(items, it2)
    return _tc_combine(users, items, item_ctx, topic_table, eu2, ei2)


# speedup vs baseline: 2.1073x; 1.0542x over previous
import functools

import jax
import jax.numpy as jnp
from jax import lax
from jax.experimental import pallas as pl
from jax.experimental.pallas import tpu as pltpu
from jax.experimental.pallas import tpu_sc as plsc

BATCH = 4096
EMBED_DIM = 64
TOPIC_SIZE = 128
TABLE_ROWS = 100000
PAIR_DIM = 2 * EMBED_DIM
SUP = 16384
HALF = SUP // 2
HBITS = 13
NSUP = -(-TABLE_ROWS // SUP)
PAIR_ROWS = NSUP * HALF

_info = plsc.get_sparse_core_info()
_NC, _NS = _info.num_cores, _info.num_subcores
_NW = _NC * _NS
_BPW = BATCH // _NW


def _tc_repack_body(t1_ref, t2_ref, out_ref):
    out_ref[...] = jnp.concatenate([t1_ref[...], t2_ref[...]], axis=0).T


def _tc_repack(table_t):
    return pl.pallas_call(
        _tc_repack_body,
        grid=(NSUP,),
        in_specs=[
            pl.BlockSpec((EMBED_DIM, HALF), lambda i: (0, 2 * i)),
            pl.BlockSpec((EMBED_DIM, HALF),
                         lambda i: (0, jnp.minimum(2 * i + 1,
                                                   TABLE_ROWS // HALF))),
        ],
        out_specs=pl.BlockSpec((HALF, PAIR_DIM), lambda i: (i, 0)),
        out_shape=jax.ShapeDtypeStruct((PAIR_ROWS, PAIR_DIM), jnp.float32),
    )(table_t, table_t)


def _sc_body(idx_hbm, tab2_hbm, out2_hbm, idx_v, pix_v, rows_v, sem):
    wid = lax.axis_index("s") * _NC + lax.axis_index("c")
    base = wid * _BPW
    pltpu.sync_copy(idx_hbm.at[pl.ds(base, _BPW)], idx_v)
    for j in range(_BPW // 16):
        r = idx_v[pl.ds(j * 16, 16)]
        pix_v[pl.ds(j * 16, 16)] = (
            ((r >> (HBITS + 1)) << HBITS) | (r & (HALF - 1)))
    pltpu.async_copy(tab2_hbm.at[pix_v], rows_v, sem).wait()
    pltpu.sync_copy(rows_v, out2_hbm.at[pl.ds(base, _BPW)])


_sc_gather = functools.partial(
    pl.kernel,
    mesh=plsc.VectorSubcoreMesh(core_axis_name="c", subcore_axis_name="s"),
    out_type=jax.ShapeDtypeStruct((BATCH, PAIR_DIM), jnp.float32),
    scratch_types=[
        pltpu.VMEM((_BPW,), jnp.int32),
        pltpu.VMEM((_BPW,), jnp.int32),
        pltpu.VMEM((_BPW, PAIR_DIM), jnp.float32),
        pltpu.SemaphoreType.DMA,
    ],
)(_sc_body)


def _tc_combine_body(users_ref, items_ref, ctx_ref, topic_ref,
                     eu2_ref, ei2_ref, out_ref):
    pu = ((users_ref[...] >> HBITS) & 1)[:, None] == 1
    pi = ((items_ref[...] >> HBITS) & 1)[:, None] == 1
    eu = jnp.where(pu, eu2_ref[:, EMBED_DIM:], eu2_ref[:, :EMBED_DIM])
    ei = jnp.where(pi, ei2_ref[:, EMBED_DIM:], ei2_ref[:, :EMBED_DIM])
    ctx = jnp.dot(ctx_ref[...], topic_ref[...],
                  preferred_element_type=jnp.float32)
    out_ref[...] = jnp.sum(eu * (ctx + ei), axis=1)


def _tc_combine(users, items, item_ctx, topic_table, eu2, ei2):
    nblk = 4
    bs = BATCH // nblk
    return pl.pallas_call(
        _tc_combine_body,
        grid=(nblk,),
        in_specs=[
            pl.BlockSpec((bs,), lambda i: (i,)),
            pl.BlockSpec((bs,), lambda i: (i,)),
            pl.BlockSpec((bs, TOPIC_SIZE), lambda i: (i, 0)),
            pl.BlockSpec((TOPIC_SIZE, EMBED_DIM), lambda i: (0, 0)),
            pl.BlockSpec((bs, PAIR_DIM), lambda i: (i, 0)),
            pl.BlockSpec((bs, PAIR_DIM), lambda i: (i, 0)),
        ],
        out_specs=pl.BlockSpec((bs,), lambda i: (i,)),
        out_shape=jax.ShapeDtypeStruct((BATCH,), jnp.float32),
    )(users, items, item_ctx, topic_table, eu2, ei2)


@jax.jit
def kernel(users, items, item_ctx, user_table, item_table, topic_table):
    ut2 = _tc_repack(user_table.T)
    eu2 = _sc_gather(users, ut2)
    it2 = _tc_repack(item_table.T)
    ei2 = _sc_gather(items, it2)
    return _tc_combine(users, items, item_ctx, topic_table, eu2, ei2)
